# baseline (speedup 1.0000x reference)
import functools
import math

import jax
import jax.numpy as jnp
from jax import lax
from jax.experimental import pallas as pl
from jax.experimental.pallas import tpu as pltpu
from jax.experimental.pallas import tpu_sc as plsc

D_MODEL = 64
SCALE = math.sqrt(D_MODEL)

NC = 2
NS = 16
NW = NC * NS
LANES = 16

CHUNK = 128
B_TOTAL = 4096 * 200
B_PER_W = B_TOTAL // NW
N_CHUNKS = B_PER_W // CHUNK


def _body(idx_hbm, table_hbm, out_hbm, idx_v, rows_v, sem):
    c = lax.axis_index("c")
    s = lax.axis_index("s")
    wid = s * NC + c
    base = wid * B_PER_W

    pltpu.sync_copy(idx_hbm.at[wid], idx_v)

    def chunk(g, carry):
        pltpu.async_copy(table_hbm.at[idx_v.at[g]], rows_v, sem).wait()

        def scale_rows(r, carry2):
            for rr in range(8):
                for d in range(D_MODEL // LANES):
                    sl = pl.ds(d * LANES, LANES)
                    rows_v[r * 8 + rr, sl] = rows_v[r * 8 + rr, sl] * SCALE
            return carry2

        lax.fori_loop(0, CHUNK // 8, scale_rows, 0, unroll=False)

        pltpu.sync_copy(rows_v, out_hbm.at[pl.ds(base + g * CHUNK, CHUNK)])
        return carry

    lax.fori_loop(0, N_CHUNKS, chunk, 0, unroll=False)


@jax.jit
def kernel(x, table):
    idx = x.reshape(NW, N_CHUNKS, CHUNK).astype(jnp.int32)
    mesh = plsc.VectorSubcoreMesh(
        core_axis_name="c", subcore_axis_name="s", num_cores=NC, num_subcores=NS
    )
    out = pl.kernel(
        _body,
        out_type=jax.ShapeDtypeStruct((B_TOTAL, D_MODEL), jnp.float32),
        mesh=mesh,
        scratch_types=[
            pltpu.VMEM((N_CHUNKS, CHUNK), jnp.int32),
            pltpu.VMEM((CHUNK, D_MODEL), jnp.float32),
            pltpu.SemaphoreType.DMA,
        ],
        compiler_params=pltpu.CompilerParams(use_tc_tiling_on_sc=False),
    )(idx, table)
    return out.reshape(4096, 200, D_MODEL)


# speedup vs baseline: 1.1629x; 1.1629x over previous
import math

import jax
import jax.numpy as jnp
from jax import lax
from jax.experimental import pallas as pl
from jax.experimental.pallas import tpu as pltpu
from jax.experimental.pallas import tpu_sc as plsc

D_MODEL = 64
SCALE = math.sqrt(D_MODEL)

NC = 2
NS = 16
NW = NC * NS
LANES = 16

IDX_W = 128
GPC = 2
C = IDX_W * GPC
B_TOTAL = 4096 * 200
B_PER_W = B_TOTAL // NW
N_IDX_ROWS = B_PER_W // IDX_W
NG = B_PER_W // C
ROWS_UNROLL = 8


def _body(idx_hbm, table_hbm, out_hbm, idx_v, raw0, raw1, scl0, scl1,
          gsem0, gsem1, osem0, osem1):
    c = lax.axis_index("c")
    s = lax.axis_index("s")
    wid = s * NC + c
    base = wid * B_PER_W
    raws = (raw0, raw1)
    scls = (scl0, scl1)
    gsems = (gsem0, gsem1)
    osems = (osem0, osem1)

    pltpu.sync_copy(idx_hbm.at[wid], idx_v)

    def gather_args(group, b, h):
        return (
            table_hbm.at[idx_v.at[group * GPC + h]],
            raws[b].at[pl.ds(h * IDX_W, IDX_W)],
            gsems[b],
        )

    def issue_gather(group, b):
        for h in range(GPC):
            pltpu.async_copy(*gather_args(group, b, h))

    def wait_gather(group, b):
        for h in range(GPC):
            pltpu.make_async_copy(*gather_args(group, b, h)).wait()

    def out_args(group, b):
        return (scls[b], out_hbm.at[pl.ds(base + group * C, C)], osems[b])

    def scale(b):
        raw = raws[b]
        scl = scls[b]

        def rowblk(r, carry):
            for rr in range(ROWS_UNROLL):
                row = r * ROWS_UNROLL + rr
                for d in range(D_MODEL // LANES):
                    sl = pl.ds(d * LANES, LANES)
                    scl[row, sl] = raw[row, sl] * SCALE
            return carry

        lax.fori_loop(0, C // ROWS_UNROLL, rowblk, 0, unroll=False)

    issue_gather(0, 0)
    issue_gather(1, 1)

    def outer(o, carry):
        for b in range(2):
            group = o * 2 + b
            wait_gather(group, b)

            @pl.when(o >= 1)
            def _():
                pltpu.make_async_copy(*out_args(group - 2, b)).wait()

            scale(b)

            @pl.when(o < (NG // 2) - 1)
            def _():
                issue_gather(group + 2, b)

            pltpu.async_copy(*out_args(group, b))
        return carry

    lax.fori_loop(0, NG // 2, outer, 0, unroll=False)

    for b in range(2):
        pltpu.make_async_copy(*out_args(NG - 2 + b, b)).wait()


@jax.jit
def kernel(x, table):
    idx = x.reshape(NW, N_IDX_ROWS, IDX_W).astype(jnp.int32)
    mesh = plsc.VectorSubcoreMesh(
        core_axis_name="c", subcore_axis_name="s", num_cores=NC, num_subcores=NS
    )
    out = pl.kernel(
        _body,
        out_type=jax.ShapeDtypeStruct((B_TOTAL, D_MODEL), jnp.float32),
        mesh=mesh,
        scratch_types=[
            pltpu.VMEM((N_IDX_ROWS, IDX_W), jnp.int32),
            pltpu.VMEM((C, D_MODEL), jnp.float32),
            pltpu.VMEM((C, D_MODEL), jnp.float32),
            pltpu.VMEM((C, D_MODEL), jnp.float32),
            pltpu.VMEM((C, D_MODEL), jnp.float32),
            pltpu.SemaphoreType.DMA,
            pltpu.SemaphoreType.DMA,
            pltpu.SemaphoreType.DMA,
            pltpu.SemaphoreType.DMA,
        ],
        compiler_params=pltpu.CompilerParams(use_tc_tiling_on_sc=False),
    )(idx, table)
    return out.reshape(4096, 200, D_MODEL)
